# baseline (speedup 1.0000x reference)
import functools

import jax
import jax.numpy as jnp
import numpy as np
from jax import lax
from jax.experimental import pallas as pl
from jax.experimental.pallas import tpu as pltpu
from jax.experimental.pallas import tpu_sc as plsc

_BOUNDS = [
    -3.0, -2.80645, -2.6129, -2.41935, -2.22581, -2.03226, -1.83871,
    -1.64516, -1.45161, -1.25806, -1.06452, -0.87097, -0.67742, -0.48387,
    -0.29032, -0.09677, 0.09677, 0.29032, 0.48387, 0.67742, 0.87097,
    1.06452, 1.25806, 1.45161, 1.64516, 1.83871, 2.03226, 2.22581,
    2.41935, 2.6129, 2.80645, 3.0,
]

N = 16777216
NC = 2
NS = 16
NW = NC * NS
PER_W = N // NW
CHUNK = 16384
NCH = PER_W // CHUNK
VECS = CHUNK // 16
SCALE = 31.0 / 6.0
EPS = 0.001
OFFSET = float(np.float32(15.5) + np.float32(EPS))

_TABLE = np.full(40, np.inf, dtype=np.float32)
_TABLE[:32] = np.asarray(_BOUNDS, dtype=np.float32)


def _body(tbl_hbm, x_hbm, out_hbm, tbl_v, in0, in1, out0, out1,
          s_in0, s_in1, s_out0, s_out1, s_tbl):
    cid = lax.axis_index("c")
    sid = lax.axis_index("s")
    wid = sid * NC + cid
    base = wid * PER_W

    pltpu.async_copy(tbl_hbm, tbl_v, s_tbl).wait()

    in_bufs = (in0, in1)
    out_bufs = (out0, out1)
    in_sems = (s_in0, s_in1)
    out_sems = (s_out0, s_out1)

    def in_copy(g, b):
        return pltpu.make_async_copy(
            x_hbm.at[pl.ds(base + g * CHUNK, CHUNK)], in_bufs[b], in_sems[b])

    def out_copy(g, b):
        return pltpu.make_async_copy(
            out_bufs[b], out_hbm.at[pl.ds(base + g * CHUNK, CHUNK)],
            out_sems[b])

    def compute(b):
        src = in_bufs[b]
        dst = out_bufs[b]

        @plsc.parallel_loop(0, CHUNK, step=16, unroll=16)
        def vec(j):
            x = src[pl.ds(j, 16)]
            t = x * SCALE + OFFSET
            t = jnp.maximum(jnp.minimum(t, 32.5), 0.0)
            c = t.astype(jnp.int32)
            bnd = plsc.load_gather(tbl_v, [c])
            res = c + (x >= bnd).astype(jnp.int32)
            dst[pl.ds(j, 16)] = res

    in_copy(0, 0).start()
    in_copy(1, 1).start()
    in_copy(0, 0).wait()
    compute(0)
    out_copy(0, 0).start()
    in_copy(2, 0).start()
    in_copy(1, 1).wait()
    compute(1)
    out_copy(1, 1).start()

    def steady(gg, carry):
        g0 = gg * 2
        in_copy(g0 + 1, 1).start()
        in_copy(g0, 0).wait()
        out_copy(g0 - 2, 0).wait()
        compute(0)
        out_copy(g0, 0).start()
        in_copy(g0 + 2, 0).start()
        in_copy(g0 + 1, 1).wait()
        out_copy(g0 - 1, 1).wait()
        compute(1)
        out_copy(g0 + 1, 1).start()
        return carry

    lax.fori_loop(1, NCH // 2 - 1, steady, 0)

    in_copy(NCH - 1, 1).start()
    in_copy(NCH - 2, 0).wait()
    out_copy(NCH - 4, 0).wait()
    compute(0)
    out_copy(NCH - 2, 0).start()
    in_copy(NCH - 1, 1).wait()
    out_copy(NCH - 3, 1).wait()
    compute(1)
    out_copy(NCH - 1, 1).start()

    out_copy(NCH - 2, 0).wait()
    out_copy(NCH - 1, 1).wait()


@functools.cache
def _make_bucketize():
    return pl.kernel(
        _body,
        out_type=jax.ShapeDtypeStruct((N,), jnp.int32),
        mesh=plsc.VectorSubcoreMesh(core_axis_name="c", subcore_axis_name="s",
                                    num_cores=NC, num_subcores=NS),
        compiler_params=pltpu.CompilerParams(needs_layout_passes=False),
        scratch_types=[
            pltpu.VMEM((40,), jnp.float32),
            pltpu.VMEM((CHUNK,), jnp.float32),
            pltpu.VMEM((CHUNK,), jnp.float32),
            pltpu.VMEM((CHUNK,), jnp.int32),
            pltpu.VMEM((CHUNK,), jnp.int32),
            pltpu.SemaphoreType.DMA,
            pltpu.SemaphoreType.DMA,
            pltpu.SemaphoreType.DMA,
            pltpu.SemaphoreType.DMA,
            pltpu.SemaphoreType.DMA,
        ],
    )


def kernel(inputs):
    tbl = jnp.asarray(_TABLE)
    return _make_bucketize()(tbl, inputs)


# speedup vs baseline: 1.0066x; 1.0066x over previous
import functools

import jax
import jax.numpy as jnp
import numpy as np
from jax import lax
from jax.experimental import pallas as pl
from jax.experimental.pallas import tpu as pltpu
from jax.experimental.pallas import tpu_sc as plsc

_BOUNDS = [
    -3.0, -2.80645, -2.6129, -2.41935, -2.22581, -2.03226, -1.83871,
    -1.64516, -1.45161, -1.25806, -1.06452, -0.87097, -0.67742, -0.48387,
    -0.29032, -0.09677, 0.09677, 0.29032, 0.48387, 0.67742, 0.87097,
    1.06452, 1.25806, 1.45161, 1.64516, 1.83871, 2.03226, 2.22581,
    2.41935, 2.6129, 2.80645, 3.0,
]

N = 16777216
NC = 2
NS = 16
NW = NC * NS
PER_W = N // NW
CHUNK = 16384
NCH = PER_W // CHUNK
VECS = CHUNK // 16
SCALE = 31.0 / 6.0
EPS = 0.001
OFFSET = float(np.float32(15.5) + np.float32(EPS))

_TABLE = np.full(40, np.inf, dtype=np.float32)
_TABLE[:32] = np.asarray(_BOUNDS, dtype=np.float32)


def _body(tbl_hbm, x_hbm, out_hbm, tbl_v, in0, in1, out0, out1,
          s_in0, s_in1, s_out0, s_out1, s_tbl):
    cid = lax.axis_index("c")
    sid = lax.axis_index("s")
    wid = sid * NC + cid
    base = wid * PER_W

    pltpu.async_copy(tbl_hbm, tbl_v, s_tbl).wait()

    in_bufs = (in0, in1)
    out_bufs = (out0, out1)
    in_sems = (s_in0, s_in1)
    out_sems = (s_out0, s_out1)

    def in_copy(g, b):
        return pltpu.make_async_copy(
            x_hbm.at[pl.ds(base + g * CHUNK, CHUNK)], in_bufs[b], in_sems[b])

    def out_copy(g, b):
        return pltpu.make_async_copy(
            out_bufs[b], out_hbm.at[pl.ds(base + g * CHUNK, CHUNK)],
            out_sems[b])

    def compute(b):
        src = in_bufs[b]
        dst = out_bufs[b]

        @plsc.parallel_loop(0, CHUNK, step=16, unroll=8)
        def vec(j):
            x = src[pl.ds(j, 16)]
            t = x * SCALE + OFFSET
            t = jnp.maximum(jnp.minimum(t, 32.5), 0.0)
            c = t.astype(jnp.int32)
            bnd = plsc.load_gather(tbl_v, [c])
            res = c + (x >= bnd).astype(jnp.int32)
            dst[pl.ds(j, 16)] = res

    in_copy(0, 0).start()
    in_copy(1, 1).start()
    in_copy(0, 0).wait()
    compute(0)
    out_copy(0, 0).start()
    in_copy(2, 0).start()
    in_copy(1, 1).wait()
    compute(1)
    out_copy(1, 1).start()

    def steady(gg, carry):
        g0 = gg * 2
        in_copy(g0 + 1, 1).start()
        in_copy(g0, 0).wait()
        out_copy(g0 - 2, 0).wait()
        compute(0)
        out_copy(g0, 0).start()
        in_copy(g0 + 2, 0).start()
        in_copy(g0 + 1, 1).wait()
        out_copy(g0 - 1, 1).wait()
        compute(1)
        out_copy(g0 + 1, 1).start()
        return carry

    lax.fori_loop(1, NCH // 2 - 1, steady, 0)

    in_copy(NCH - 1, 1).start()
    in_copy(NCH - 2, 0).wait()
    out_copy(NCH - 4, 0).wait()
    compute(0)
    out_copy(NCH - 2, 0).start()
    in_copy(NCH - 1, 1).wait()
    out_copy(NCH - 3, 1).wait()
    compute(1)
    out_copy(NCH - 1, 1).start()

    out_copy(NCH - 2, 0).wait()
    out_copy(NCH - 1, 1).wait()


@functools.cache
def _make_bucketize():
    return pl.kernel(
        _body,
        out_type=jax.ShapeDtypeStruct((N,), jnp.int32),
        mesh=plsc.VectorSubcoreMesh(core_axis_name="c", subcore_axis_name="s",
                                    num_cores=NC, num_subcores=NS),
        compiler_params=pltpu.CompilerParams(needs_layout_passes=False),
        scratch_types=[
            pltpu.VMEM((40,), jnp.float32),
            pltpu.VMEM((CHUNK,), jnp.float32),
            pltpu.VMEM((CHUNK,), jnp.float32),
            pltpu.VMEM((CHUNK,), jnp.int32),
            pltpu.VMEM((CHUNK,), jnp.int32),
            pltpu.SemaphoreType.DMA,
            pltpu.SemaphoreType.DMA,
            pltpu.SemaphoreType.DMA,
            pltpu.SemaphoreType.DMA,
            pltpu.SemaphoreType.DMA,
        ],
    )


def kernel(inputs):
    tbl = jnp.asarray(_TABLE)
    return _make_bucketize()(tbl, inputs)


# speedup vs baseline: 1.0141x; 1.0074x over previous
import functools

import jax
import jax.numpy as jnp
import numpy as np
from jax import lax
from jax.experimental import pallas as pl
from jax.experimental.pallas import tpu as pltpu
from jax.experimental.pallas import tpu_sc as plsc

_BOUNDS = [
    -3.0, -2.80645, -2.6129, -2.41935, -2.22581, -2.03226, -1.83871,
    -1.64516, -1.45161, -1.25806, -1.06452, -0.87097, -0.67742, -0.48387,
    -0.29032, -0.09677, 0.09677, 0.29032, 0.48387, 0.67742, 0.87097,
    1.06452, 1.25806, 1.45161, 1.64516, 1.83871, 2.03226, 2.22581,
    2.41935, 2.6129, 2.80645, 3.0,
]

N = 16777216
NC = 2
NS = 16
NW = NC * NS
PER_W = N // NW
CHUNK = 16384
NCH = PER_W // CHUNK
VECS = CHUNK // 16
SCALE = 31.0 / 6.0
EPS = 0.001
OFFSET = float(np.float32(15.5) + np.float32(EPS))

_TABLE = np.full(40, np.inf, dtype=np.float32)
_TABLE[:32] = np.asarray(_BOUNDS, dtype=np.float32)


def _body(tbl_hbm, x_hbm, out_hbm, tbl_v, in0, in1, out0, out1,
          s_in0, s_in1, s_out0, s_out1, s_tbl):
    cid = lax.axis_index("c")
    sid = lax.axis_index("s")
    wid = sid * NC + cid
    base = wid * PER_W

    tbl_copy = pltpu.make_async_copy(tbl_hbm, tbl_v, s_tbl)
    tbl_copy.start()

    in_bufs = (in0, in1)
    out_bufs = (out0, out1)
    in_sems = (s_in0, s_in1)
    out_sems = (s_out0, s_out1)

    def in_copy(g, b):
        return pltpu.make_async_copy(
            x_hbm.at[pl.ds(base + g * CHUNK, CHUNK)], in_bufs[b], in_sems[b])

    def out_copy(g, b):
        return pltpu.make_async_copy(
            out_bufs[b], out_hbm.at[pl.ds(base + g * CHUNK, CHUNK)],
            out_sems[b])

    def compute(b):
        src = in_bufs[b]
        dst = out_bufs[b]

        @plsc.parallel_loop(0, CHUNK, step=16, unroll=8)
        def vec(j):
            x = src[pl.ds(j, 16)]
            t = x * SCALE + OFFSET
            t = jnp.maximum(jnp.minimum(t, 32.5), 0.0)
            c = t.astype(jnp.int32)
            bnd = plsc.load_gather(tbl_v, [c])
            res = c + (x >= bnd).astype(jnp.int32)
            dst[pl.ds(j, 16)] = res

    in_copy(0, 0).start()
    in_copy(1, 1).start()
    tbl_copy.wait()
    in_copy(0, 0).wait()
    compute(0)
    out_copy(0, 0).start()
    in_copy(2, 0).start()
    in_copy(1, 1).wait()
    compute(1)
    out_copy(1, 1).start()

    def steady(gg, carry):
        g0 = gg * 2
        in_copy(g0 + 1, 1).start()
        in_copy(g0, 0).wait()
        out_copy(g0 - 2, 0).wait()
        compute(0)
        out_copy(g0, 0).start()
        in_copy(g0 + 2, 0).start()
        in_copy(g0 + 1, 1).wait()
        out_copy(g0 - 1, 1).wait()
        compute(1)
        out_copy(g0 + 1, 1).start()
        return carry

    lax.fori_loop(1, NCH // 2 - 1, steady, 0)

    in_copy(NCH - 1, 1).start()
    in_copy(NCH - 2, 0).wait()
    out_copy(NCH - 4, 0).wait()
    compute(0)
    out_copy(NCH - 2, 0).start()
    in_copy(NCH - 1, 1).wait()
    out_copy(NCH - 3, 1).wait()
    compute(1)
    out_copy(NCH - 1, 1).start()

    out_copy(NCH - 2, 0).wait()
    out_copy(NCH - 1, 1).wait()


@functools.cache
def _make_bucketize():
    return pl.kernel(
        _body,
        out_type=jax.ShapeDtypeStruct((N,), jnp.int32),
        mesh=plsc.VectorSubcoreMesh(core_axis_name="c", subcore_axis_name="s",
                                    num_cores=NC, num_subcores=NS),
        compiler_params=pltpu.CompilerParams(needs_layout_passes=False),
        scratch_types=[
            pltpu.VMEM((40,), jnp.float32),
            pltpu.VMEM((CHUNK,), jnp.float32),
            pltpu.VMEM((CHUNK,), jnp.float32),
            pltpu.VMEM((CHUNK,), jnp.int32),
            pltpu.VMEM((CHUNK,), jnp.int32),
            pltpu.SemaphoreType.DMA,
            pltpu.SemaphoreType.DMA,
            pltpu.SemaphoreType.DMA,
            pltpu.SemaphoreType.DMA,
            pltpu.SemaphoreType.DMA,
        ],
    )


def kernel(inputs):
    tbl = jnp.asarray(_TABLE)
    return _make_bucketize()(tbl, inputs)
